# Initial kernel scaffold; baseline (speedup 1.0000x reference)
import jax
import jax.numpy as jnp
from jax.experimental import pallas as pl


def kernel(x, time_indices, value_W, value_b, daily_tab0, daily_tab1, daily_tab2, daily_tab3, daily_tab4, daily_W, daily_b, phase_embed, pos_W, pos_b, neg_W, neg_b, vq_W, vq_b, vq_cW, vq_cb):
    raise NotImplementedError("write your pallas kernel here")


# speedup vs baseline: 1.5340x; 1.5340x over previous
import functools

import numpy as np
import jax
import jax.numpy as jnp
from jax import lax
from jax.experimental import pallas as pl
from jax.experimental.pallas import tpu as pltpu
from jax.experimental.pallas import tpu_sc as plsc

D_MODEL = 512
SPD = 86400
PERIODS = (SPD, SPD // 2, SPD // 3, SPD // 4, SPD // 6)
NUM_TABLES = 5
SUB = D_MODEL // NUM_TABLES

_NC = 2
_NS = 16
_NW = _NC * _NS
_L = 16
_GCHUNK = 128


def _make_pe(d_model=D_MODEL, max_len=5000, power_freq=50.0, sample_rate=1.0):
    pe = np.zeros((max_len, d_model), dtype=np.float32)
    pos = np.arange(max_len, dtype=np.float32)
    harmonics = [1, 2, 3, 5, 7]
    hd = d_model // (len(harmonics) * 2)
    for h_idx, h in enumerate(harmonics):
        omega = 2.0 * np.pi * power_freq * h / sample_rate
        start = h_idx * hd * 2
        end = min(start + hd * 2, d_model)
        for i in range(0, end - start, 2):
            ps = i * np.pi / (end - start)
            if start + i < d_model:
                pe[:, start + i] = np.sin(pos * omega + ps)
            if start + i + 1 < d_model:
                pe[:, start + i + 1] = np.cos(pos * omega + ps)
    return pe


_PE = _make_pe()


_TAB_OFF = tuple(int(np.cumsum([0] + list(PERIODS))[i]) for i in range(NUM_TABLES))
_DPAD = 128


def _sc_gather(time_idx, tab_all):
    tot = time_idx.shape[0]
    b_per_w = tot // _NW
    nch = b_per_w // _GCHUNK
    mesh = plsc.VectorSubcoreMesh(core_axis_name="c", subcore_axis_name="s")

    @functools.partial(
        pl.kernel,
        out_type=jax.ShapeDtypeStruct((NUM_TABLES, tot, _DPAD), jnp.float32),
        mesh=mesh,
        scratch_types=[
            pltpu.VMEM((b_per_w,), jnp.int32),
            pltpu.VMEM((nch, _GCHUNK), jnp.int32),
            pltpu.VMEM((b_per_w, _DPAD), jnp.float32),
            pltpu.SemaphoreType.DMA,
        ],
    )
    def k(idx_hbm, tab_hbm, out_hbm, idx_raw, idx_mod, rows, sem):
        wid = lax.axis_index("s") * _NC + lax.axis_index("c")
        base = wid * b_per_w
        pltpu.sync_copy(idx_hbm.at[pl.ds(base, b_per_w)], idx_raw)
        for i, p in enumerate(PERIODS):
            off = jnp.int32(_TAB_OFF[i])

            def chunk_body(c, _, p=p, off=off):
                for j in range(_GCHUNK // _L):
                    v = idx_raw[pl.ds(c * _GCHUNK + j * _L, _L)]
                    idx_mod[c, pl.ds(j * _L, _L)] = lax.rem(v, jnp.int32(p)) + off
                pltpu.async_copy(
                    tab_hbm.at[idx_mod.at[c]],
                    rows.at[pl.ds(c * _GCHUNK, _GCHUNK)],
                    sem,
                ).wait()
                return 0

            lax.fori_loop(0, nch, chunk_body, 0)
            pltpu.sync_copy(rows, out_hbm.at[i, pl.ds(base, b_per_w)])

    return k(time_idx, tab_all)


def _tc_combine(g, x2d, pe, wd, wx, bias, interpret=False):
    tot, c = x2d.shape
    t_len = pe.shape[0]
    bt = 512
    nblk = tot // bt
    tpb = t_len // bt

    def body(g_ref, x_ref, pe_ref, wd_ref, wx_ref, b_ref, o_ref):
        acc = jnp.dot(x_ref[...], wx_ref[...], preferred_element_type=jnp.float32)
        for i in range(NUM_TABLES):
            acc += jnp.dot(g_ref[i], wd_ref[i], preferred_element_type=jnp.float32)
        o_ref[...] = acc + pe_ref[...] + b_ref[...]

    return pl.pallas_call(
        body,
        grid=(nblk,),
        in_specs=[
            pl.BlockSpec((NUM_TABLES, bt, _DPAD), lambda i: (0, i, 0)),
            pl.BlockSpec((bt, c), lambda i: (i, 0)),
            pl.BlockSpec((bt, D_MODEL), lambda i: (i % tpb, 0)),
            pl.BlockSpec((NUM_TABLES, _DPAD, D_MODEL), lambda i: (0, 0, 0)),
            pl.BlockSpec((c, D_MODEL), lambda i: (0, 0)),
            pl.BlockSpec((1, D_MODEL), lambda i: (0, 0)),
        ],
        out_specs=pl.BlockSpec((bt, D_MODEL), lambda i: (i, 0)),
        out_shape=jax.ShapeDtypeStruct((tot, D_MODEL), jnp.float32),
        interpret=interpret,
    )(g, x2d, pe, wd, wx, bias)


def _finish(g, x, value_W, value_b, daily_W, daily_b, phase_embed, pos_W,
            pos_b, neg_W, neg_b, vq_W, vq_b, vq_cW, vq_cb, interpret=False):
    B, T, C = x.shape
    nq = vq_W.shape[0]
    qv = vq_cW[:, :nq] @ vq_W[:, 0]
    qc = vq_cW[:, :nq] @ vq_b + vq_cb
    negp = neg_W[:, jnp.array([0, 2, 1])]
    wx = (value_W + pos_W + 0.1 * negp + qv[:, None] / (3.0 * 220.0)).T
    bias = (value_b + daily_b + pos_b + 0.1 * neg_b + phase_embed.mean(0)
            + qc - qv)[None, :]
    wd = jnp.zeros((NUM_TABLES, _DPAD, D_MODEL), jnp.float32)
    wd = wd.at[:, :SUB, :].set(daily_W.T.reshape(NUM_TABLES, SUB, D_MODEL))
    pe = jnp.asarray(_PE[:T])
    out = _tc_combine(g, x.reshape(B * T, C), pe, wd, wx, bias,
                      interpret=interpret)
    return out.reshape(B, T, D_MODEL)


def kernel(x, time_indices, value_W, value_b, daily_tab0, daily_tab1,
           daily_tab2, daily_tab3, daily_tab4, daily_W, daily_b, phase_embed,
           pos_W, pos_b, neg_W, neg_b, vq_W, vq_b, vq_cW, vq_cb):
    B, T, C = x.shape
    ti = time_indices.reshape(B * T).astype(jnp.int32)
    tabs = (daily_tab0, daily_tab1, daily_tab2, daily_tab3, daily_tab4)
    tab_all = jnp.concatenate(
        [jnp.pad(t, ((0, 0), (0, _DPAD - SUB))) for t in tabs], axis=0)
    g = _sc_gather(ti, tab_all)
    return _finish(g, x, value_W, value_b, daily_W, daily_b, phase_embed,
                   pos_W, pos_b, neg_W, neg_b, vq_W, vq_b, vq_cW, vq_cb)
